# baseline (speedup 1.0000x reference)
import functools

import jax
import jax.numpy as jnp
from jax.experimental import pallas as pl

NA = 50000
NT = 50000
E = 800000
H = 64

ENC_R = 2000
POST_R = 2000


def _enc_body(x_ref, w_ref, b_ref, g_ref, be_ref, h_ref, gtab_ref):
    x = x_ref[...]
    y = jnp.dot(x, w_ref[...].T, preferred_element_type=jnp.float32)
    y = y + b_ref[...]
    m = jnp.mean(y, axis=-1, keepdims=True)
    v = jnp.mean((y - m) ** 2, axis=-1, keepdims=True)
    y = (y - m) * jax.lax.rsqrt(v + 1e-5) * g_ref[...] + be_ref[...]
    h = jnp.maximum(y, 0.0)
    h_ref[...] = h
    ones = jnp.ones((h.shape[0], 1), jnp.float32)
    zeros = jnp.zeros((h.shape[0], 3), jnp.float32)
    g0 = jnp.concatenate([h[:, :32], ones, zeros], axis=1)
    g1 = jnp.concatenate([h[:, 32:], ones, zeros], axis=1)
    gtab_ref[...] = jnp.stack([g0, g1], axis=0)


def _encode(x, w, b, g, be, n):
    grid = (n // ENC_R,)
    return pl.pallas_call(
        _enc_body,
        grid=grid,
        in_specs=[
            pl.BlockSpec((ENC_R, 128), lambda i: (i, 0)),
            pl.BlockSpec((H, 128), lambda i: (0, 0)),
            pl.BlockSpec((H,), lambda i: (0,)),
            pl.BlockSpec((H,), lambda i: (0,)),
            pl.BlockSpec((H,), lambda i: (0,)),
        ],
        out_specs=[
            pl.BlockSpec((ENC_R, H), lambda i: (i, 0)),
            pl.BlockSpec((2, ENC_R, 36), lambda i: (0, i, 0)),
        ],
        out_shape=[
            jax.ShapeDtypeStruct((n, H), jnp.float32),
            jax.ShapeDtypeStruct((2, n, 36), jnp.float32),
        ],
    )(x, w, b, g, be)


def _post_body(st_ref, sa_ref, ha_ref, ht_ref, wlst_ref, blst_ref, wrst_ref,
               wlts_ref, blts_ref, wrts_ref, wh1_ref, bh1_ref, a_ref, t_ref):
    st = st_ref[...]
    sa = sa_ref[...]
    cnt_t = jnp.maximum(st[0, :, 32:33], 1.0)
    cnt_a = jnp.maximum(sa[0, :, 32:33], 1.0)
    agg_t = jnp.concatenate([st[0, :, :32], st[1, :, :32]], axis=1) / cnt_t
    agg_a = jnp.concatenate([sa[0, :, :32], sa[1, :, :32]], axis=1) / cnt_a
    new_t = (jnp.dot(agg_t, wlst_ref[...].T, preferred_element_type=jnp.float32)
             + blst_ref[...]
             + jnp.dot(ht_ref[...], wrst_ref[...].T, preferred_element_type=jnp.float32))
    new_a = (jnp.dot(agg_a, wlts_ref[...].T, preferred_element_type=jnp.float32)
             + blts_ref[...]
             + jnp.dot(ha_ref[...], wrts_ref[...].T, preferred_element_type=jnp.float32))
    wh1 = wh1_ref[...]
    a_ref[...] = (jnp.dot(new_a, wh1[:, :H].T, preferred_element_type=jnp.float32)
                  + bh1_ref[...])
    t_ref[...] = jnp.dot(new_t, wh1[:, H:].T, preferred_element_type=jnp.float32)


def _post(s_t, s_a, h_a, h_t, wlst, blst, wrst, wlts, blts, wrts, wh1, bh1):
    grid = (NA // POST_R,)
    return pl.pallas_call(
        _post_body,
        grid=grid,
        in_specs=[
            pl.BlockSpec((2, POST_R, 36), lambda i: (0, i, 0)),
            pl.BlockSpec((2, POST_R, 36), lambda i: (0, i, 0)),
            pl.BlockSpec((POST_R, H), lambda i: (i, 0)),
            pl.BlockSpec((POST_R, H), lambda i: (i, 0)),
            pl.BlockSpec((H, H), lambda i: (0, 0)),
            pl.BlockSpec((H,), lambda i: (0,)),
            pl.BlockSpec((H, H), lambda i: (0, 0)),
            pl.BlockSpec((H, H), lambda i: (0, 0)),
            pl.BlockSpec((H,), lambda i: (0,)),
            pl.BlockSpec((H, H), lambda i: (0, 0)),
            pl.BlockSpec((H, 2 * H), lambda i: (0, 0)),
            pl.BlockSpec((H,), lambda i: (0,)),
        ],
        out_specs=[
            pl.BlockSpec((POST_R, H), lambda i: (i, 0)),
            pl.BlockSpec((POST_R, H), lambda i: (i, 0)),
        ],
        out_shape=[
            jax.ShapeDtypeStruct((NA, H), jnp.float32),
            jax.ShapeDtypeStruct((NT, H), jnp.float32),
        ],
    )(s_t, s_a, h_a, h_t, wlst, blst, wrst, wlts, blts, wrts, wh1, bh1)


def kernel(x_agent, x_task, W_enc_a, b_enc_a, g_ln_a, be_ln_a, W_enc_t,
           b_enc_t, g_ln_t, be_ln_t, Wl_st, bl_st, Wr_st, Wl_ts, bl_ts,
           Wr_ts, W_h1, b_h1, W_h2, b_h2, edge_index):
    h_a, g_a = _encode(x_agent, W_enc_a, b_enc_a, g_ln_a, be_ln_a, NA)
    h_t, g_t = _encode(x_task, W_enc_t, b_enc_t, g_ln_t, be_ln_t, NT)
    src = edge_index[0]
    dst = edge_index[1]
    g_a_flat = g_a.reshape(2 * NA, 36)
    g_t_flat = g_t.reshape(2 * NT, 36)
    src2 = jnp.concatenate([src, src + NA])
    dst2 = jnp.concatenate([dst, dst + NT])
    s_t = jax.ops.segment_sum(g_a_flat[src2], dst2, num_segments=2 * NT)
    s_t = s_t.reshape(2, NT, 36)
    s_a = jax.ops.segment_sum(g_t_flat[dst2], src2, num_segments=2 * NA)
    s_a = s_a.reshape(2, NA, 36)
    a_tab, t_tab = _post(s_t, s_a, h_a, h_t, Wl_st, bl_st, Wr_st,
                         Wl_ts, bl_ts, Wr_ts, W_h1, b_h1)
    z = jnp.maximum(a_tab[src] + t_tab[dst], 0.0)
    logits = z @ W_h2[0] + b_h2[0]
    return logits


# speedup vs baseline: 5.9383x; 5.9383x over previous
import functools

import jax
import jax.numpy as jnp
from jax import lax
from jax.experimental import pallas as pl
from jax.experimental.pallas import tpu as pltpu
from jax.experimental.pallas import tpu_sc as plsc

NA = 50000
NT = 50000
E = 800000
H = 64

RW = 32
CW = 16
NP = 50008
ACC_R = 50048
TPT = ACC_R // 16
E_PAD = 802816
EROWS = E_PAD // 128
TILE_EROWS = EROWS // 16
EH_TILE_EROWS = EROWS // 32

FKB = 2
FCHUNK = FKB * 128
NFCH = TILE_EROWS // FKB

CKB = 8
CCHUNK = CKB * 128
NCCH = TILE_EROWS // CKB

EKB = 2
ECHUNK = EKB * 128
NECH = EH_TILE_EROWS // EKB

ENC_R = 2000
POST_R = 2000

_SC_PARAMS = pltpu.CompilerParams(use_tc_tiling_on_sc=False,
                                  needs_layout_passes=False)


def _enc_body(x_ref, w_ref, b_ref, g_ref, be_ref, h_ref, gtab_ref):
    x = x_ref[...]
    y = jnp.dot(x, w_ref[...].T, preferred_element_type=jnp.float32)
    y = y + b_ref[...]
    m = jnp.mean(y, axis=-1, keepdims=True)
    v = jnp.mean((y - m) ** 2, axis=-1, keepdims=True)
    y = (y - m) * lax.rsqrt(v + 1e-5) * g_ref[...] + be_ref[...]
    h = jnp.maximum(y, 0.0)
    h_ref[...] = h
    gtab_ref[...] = jnp.stack([h[:, :RW], h[:, RW:]], axis=0)


def _encode(x, w, b, g, be, n):
    return pl.pallas_call(
        _enc_body,
        grid=(n // ENC_R,),
        in_specs=[
            pl.BlockSpec((ENC_R, 128), lambda i: (i, 0)),
            pl.BlockSpec((H, 128), lambda i: (0, 0)),
            pl.BlockSpec((H,), lambda i: (0,)),
            pl.BlockSpec((H,), lambda i: (0,)),
            pl.BlockSpec((H,), lambda i: (0,)),
        ],
        out_specs=[
            pl.BlockSpec((ENC_R, H), lambda i: (i, 0)),
            pl.BlockSpec((2, ENC_R, RW), lambda i: (0, i, 0)),
        ],
        out_shape=[
            jax.ShapeDtypeStruct((n, H), jnp.float32),
            jax.ShapeDtypeStruct((2, n, RW), jnp.float32),
        ],
    )(x, w, b, g, be)


def _cnt_body(raws, ones_hbm, zc_hbm, cnt_out, acc, sidx0, sidx1, ones_v, sem):
    c = lax.axis_index("c")
    s = lax.axis_index("s")
    pltpu.sync_copy(zc_hbm, acc.at[pl.ds(s * TPT, TPT)])
    pltpu.sync_copy(ones_hbm, ones_v)
    plsc.subcore_barrier()
    base = s * TILE_EROWS

    def stage(k, buf):
        pltpu.sync_copy(raws.at[c, pl.ds(base + k * CKB, CKB)], buf)

    def scat(buf):
        for j in range(CKB):
            pltpu.sync_copy(ones_v.at[pl.ds(j * 128, 128)],
                            acc.at[buf.at[j]], add=True)

    stage(0, sidx0)

    def body(i, _):
        stage(2 * i + 1, sidx1)
        scat(sidx0)
        stage(2 * i + 2, sidx0)
        scat(sidx1)
        return _

    lax.fori_loop(0, (NCCH - 1) // 2, body, None)
    scat(sidx0)
    plsc.subcore_barrier()
    pltpu.sync_copy(acc.at[pl.ds(s * TPT, TPT)],
                    cnt_out.at[c, pl.ds(s * TPT, TPT)])


def _counts(raws, ones_hbm, zc_hbm):
    mesh = plsc.VectorSubcoreMesh(core_axis_name="c", subcore_axis_name="s")
    f = functools.partial(
        pl.kernel,
        out_type=jax.ShapeDtypeStruct((2, ACC_R, CW), jnp.float32),
        mesh=mesh,
        compiler_params=_SC_PARAMS,
        scratch_types=[
            pltpu.VMEM_SHARED((ACC_R, CW), jnp.float32),
            pltpu.VMEM((CKB, 128), jnp.int32),
            pltpu.VMEM((CKB, 128), jnp.int32),
            pltpu.VMEM((CCHUNK, CW), jnp.float32),
            pltpu.SemaphoreType.DMA,
        ],
    )(_cnt_body)
    return f(raws, ones_hbm, zc_hbm)


def _feat_phase(c, s, table, goff, sraw, out, acc, gidx0, gidx1, sidx0, sidx1,
                rows0, rows1, sem0, sem1, zf_hbm):
    pltpu.sync_copy(zf_hbm, acc.at[pl.ds(s * TPT, TPT)])
    plsc.subcore_barrier()
    base = s * TILE_EROWS

    def stage_issue(k, gbuf, sbuf, rbuf, sem):
        r0 = base + k * FKB
        pltpu.sync_copy(goff.at[c, pl.ds(r0, FKB)], gbuf)
        pltpu.sync_copy(sraw.at[pl.ds(r0, FKB)], sbuf)
        for j in range(FKB):
            pltpu.async_copy(table.at[gbuf.at[j]],
                             rbuf.at[pl.ds(j * 128, 128)], sem)

    def wait(rbuf, sem):
        for j in range(FKB):
            pltpu.make_async_copy(table.at[pl.ds(0, 128)],
                                  rbuf.at[pl.ds(j * 128, 128)], sem).wait()

    def scat(rbuf, sbuf):
        for j in range(FKB):
            pltpu.sync_copy(rbuf.at[pl.ds(j * 128, 128)],
                            acc.at[sbuf.at[j]], add=True)

    stage_issue(0, gidx0, sidx0, rows0, sem0)

    def body(i, _):
        stage_issue(2 * i + 1, gidx1, sidx1, rows1, sem1)
        wait(rows0, sem0)
        scat(rows0, sidx0)
        stage_issue(2 * i + 2, gidx0, sidx0, rows0, sem0)
        wait(rows1, sem1)
        scat(rows1, sidx1)
        return _

    lax.fori_loop(0, (NFCH - 2) // 2, body, None)
    stage_issue(NFCH - 1, gidx1, sidx1, rows1, sem1)
    wait(rows0, sem0)
    scat(rows0, sidx0)
    wait(rows1, sem1)
    scat(rows1, sidx1)
    plsc.subcore_barrier()
    pltpu.sync_copy(acc.at[pl.ds(s * TPT, TPT)],
                    out.at[c, pl.ds(s * TPT, TPT)])
    plsc.subcore_barrier()


def _feat_body(g_a, g_t, src_off, dst_off, dst_raw, src_raw, zf_hbm,
               s_t, s_a, acc, gidx0, gidx1, sidx0, sidx1, rows0, rows1,
               sem0, sem1):
    c = lax.axis_index("c")
    s = lax.axis_index("s")
    _feat_phase(c, s, g_a, src_off, dst_raw, s_t, acc, gidx0, gidx1,
                sidx0, sidx1, rows0, rows1, sem0, sem1, zf_hbm)
    _feat_phase(c, s, g_t, dst_off, src_raw, s_a, acc, gidx0, gidx1,
                sidx0, sidx1, rows0, rows1, sem0, sem1, zf_hbm)


def _feat(g_a, g_t, src_off, dst_off, dst_raw, src_raw, zf_hbm):
    mesh = plsc.VectorSubcoreMesh(core_axis_name="c", subcore_axis_name="s")
    f = functools.partial(
        pl.kernel,
        out_type=[
            jax.ShapeDtypeStruct((2, ACC_R, RW), jnp.float32),
            jax.ShapeDtypeStruct((2, ACC_R, RW), jnp.float32),
        ],
        mesh=mesh,
        compiler_params=_SC_PARAMS,
        scratch_types=[
            pltpu.VMEM_SHARED((ACC_R, RW), jnp.float32),
            pltpu.VMEM((FKB, 128), jnp.int32),
            pltpu.VMEM((FKB, 128), jnp.int32),
            pltpu.VMEM((FKB, 128), jnp.int32),
            pltpu.VMEM((FKB, 128), jnp.int32),
            pltpu.VMEM((FCHUNK, RW), jnp.float32),
            pltpu.VMEM((FCHUNK, RW), jnp.float32),
            pltpu.SemaphoreType.DMA,
            pltpu.SemaphoreType.DMA,
        ],
    )(_feat_body)
    return f(g_a, g_t, src_off, dst_off, dst_raw, src_raw, zf_hbm)


def _post_body(st_ref, sa_ref, cnt_ref, ha_ref, ht_ref, wlst_ref, blst_ref,
               wrst_ref, wlts_ref, blts_ref, wrts_ref, wh1_ref, bh1_ref,
               a_ref, t_ref):
    st = st_ref[...]
    sa = sa_ref[...]
    cnt = cnt_ref[...]
    cnt_t = jnp.maximum(cnt[0, :, 0:1], 1.0)
    cnt_a = jnp.maximum(cnt[1, :, 0:1], 1.0)
    agg_t = jnp.concatenate([st[0], st[1]], axis=1) / cnt_t
    agg_a = jnp.concatenate([sa[0], sa[1]], axis=1) / cnt_a
    new_t = (jnp.dot(agg_t, wlst_ref[...].T, preferred_element_type=jnp.float32)
             + blst_ref[...]
             + jnp.dot(ht_ref[...], wrst_ref[...].T, preferred_element_type=jnp.float32))
    new_a = (jnp.dot(agg_a, wlts_ref[...].T, preferred_element_type=jnp.float32)
             + blts_ref[...]
             + jnp.dot(ha_ref[...], wrts_ref[...].T, preferred_element_type=jnp.float32))
    wh1 = wh1_ref[...]
    a_ref[...] = (jnp.dot(new_a, wh1[:, :H].T, preferred_element_type=jnp.float32)
                  + bh1_ref[...])
    t_ref[...] = jnp.dot(new_t, wh1[:, H:].T, preferred_element_type=jnp.float32)


def _post(s_t, s_a, cnt, h_a, h_t, wlst, blst, wrst, wlts, blts, wrts,
          wh1, bh1):
    return pl.pallas_call(
        _post_body,
        grid=(NA // POST_R,),
        in_specs=[
            pl.BlockSpec((2, POST_R, RW), lambda i: (0, i, 0)),
            pl.BlockSpec((2, POST_R, RW), lambda i: (0, i, 0)),
            pl.BlockSpec((2, POST_R, CW), lambda i: (0, i, 0)),
            pl.BlockSpec((POST_R, H), lambda i: (i, 0)),
            pl.BlockSpec((POST_R, H), lambda i: (i, 0)),
            pl.BlockSpec((H, H), lambda i: (0, 0)),
            pl.BlockSpec((H,), lambda i: (0,)),
            pl.BlockSpec((H, H), lambda i: (0, 0)),
            pl.BlockSpec((H, H), lambda i: (0, 0)),
            pl.BlockSpec((H,), lambda i: (0,)),
            pl.BlockSpec((H, H), lambda i: (0, 0)),
            pl.BlockSpec((H, 2 * H), lambda i: (0, 0)),
            pl.BlockSpec((H,), lambda i: (0,)),
        ],
        out_specs=[
            pl.BlockSpec((POST_R, H), lambda i: (i, 0)),
            pl.BlockSpec((POST_R, H), lambda i: (i, 0)),
        ],
        out_shape=[
            jax.ShapeDtypeStruct((NA, H), jnp.float32),
            jax.ShapeDtypeStruct((NT, H), jnp.float32),
        ],
    )(s_t, s_a, cnt, h_a, h_t, wlst, blst, wrst, wlts, blts, wrts, wh1, bh1)


def _edge_body(a_tab, t_tab, eidx, w2_hbm, b2_hbm, logits,
               aidx0, aidx1, tidx0, tidx1, arows0, arows1, trows0, trows1,
               out0, out1, w2_v, b2_v, sg0, sg1, sw0, sw1):
    c = lax.axis_index("c")
    s = lax.axis_index("s")
    wid = s * 2 + c
    base = wid * EH_TILE_EROWS
    obase = wid * (EH_TILE_EROWS * 128)
    pltpu.sync_copy(w2_hbm, w2_v)
    pltpu.sync_copy(b2_hbm, b2_v)
    pltpu.async_copy(out0, logits.at[pl.ds(E, ECHUNK)], sw0)
    pltpu.async_copy(out1, logits.at[pl.ds(E, ECHUNK)], sw1)

    def stage_issue(k, aidx, tidx, arows, trows, sem):
        r0 = base + k * EKB
        pltpu.sync_copy(eidx.at[0, pl.ds(r0, EKB)], aidx)
        pltpu.sync_copy(eidx.at[1, pl.ds(r0, EKB)], tidx)
        for j in range(EKB):
            pltpu.async_copy(a_tab.at[aidx.at[j]],
                             arows.at[pl.ds(j * 128, 128)], sem)
            pltpu.async_copy(t_tab.at[tidx.at[j]],
                             trows.at[pl.ds(j * 128, 128)], sem)

    def wait_pair(arows, trows, sem):
        for j in range(EKB):
            pltpu.make_async_copy(a_tab.at[pl.ds(0, 128)],
                                  arows.at[pl.ds(j * 128, 128)], sem).wait()
            pltpu.make_async_copy(t_tab.at[pl.ds(0, 128)],
                                  trows.at[pl.ds(j * 128, 128)], sem).wait()

    iot = lax.iota(jnp.int32, 16)

    def compute(k, arows, trows, out, sw):
        pltpu.make_async_copy(out, logits.at[pl.ds(E, ECHUNK)], sw).wait()

        def group(g, _):
            row_idx = iot + g * 16
            acc = b2_v[...]

            def fblk(fb, acc):
                for u in range(8):
                    f = fb * 8 + u
                    colf = jnp.full((16,), f, jnp.int32)
                    av = plsc.load_gather(arows, [row_idx, colf])
                    tv = plsc.load_gather(trows, [row_idx, colf])
                    acc = acc + jnp.maximum(av + tv, 0.0) * w2_v[f]
                return acc

            acc = lax.fori_loop(0, 8, fblk, acc)
            out[pl.ds(g * 16, 16)] = acc
            return _

        lax.fori_loop(0, ECHUNK // 16, group, None)
        pltpu.async_copy(out, logits.at[pl.ds(obase + k * ECHUNK, ECHUNK)], sw)

    stage_issue(0, aidx0, tidx0, arows0, trows0, sg0)

    def body(i, _):
        stage_issue(2 * i + 1, aidx1, tidx1, arows1, trows1, sg1)
        wait_pair(arows0, trows0, sg0)
        compute(2 * i, arows0, trows0, out0, sw0)
        stage_issue(2 * i + 2, aidx0, tidx0, arows0, trows0, sg0)
        wait_pair(arows1, trows1, sg1)
        compute(2 * i + 1, arows1, trows1, out1, sw1)
        return _

    lax.fori_loop(0, (NECH - 2) // 2, body, None)
    stage_issue(NECH - 1, aidx1, tidx1, arows1, trows1, sg1)
    wait_pair(arows0, trows0, sg0)
    compute(NECH - 2, arows0, trows0, out0, sw0)
    wait_pair(arows1, trows1, sg1)
    compute(NECH - 1, arows1, trows1, out1, sw1)
    pltpu.make_async_copy(out0, logits.at[pl.ds(E, ECHUNK)], sw0).wait()
    pltpu.make_async_copy(out1, logits.at[pl.ds(E, ECHUNK)], sw1).wait()


def _edge_head(a_tab, t_tab, eidx, w2b, b2b):
    mesh = plsc.VectorSubcoreMesh(core_axis_name="c", subcore_axis_name="s")
    f = functools.partial(
        pl.kernel,
        out_type=jax.ShapeDtypeStruct((E_PAD,), jnp.float32),
        mesh=mesh,
        compiler_params=_SC_PARAMS,
        scratch_types=[
            pltpu.VMEM((EKB, 128), jnp.int32),
            pltpu.VMEM((EKB, 128), jnp.int32),
            pltpu.VMEM((EKB, 128), jnp.int32),
            pltpu.VMEM((EKB, 128), jnp.int32),
            pltpu.VMEM((ECHUNK, H), jnp.float32),
            pltpu.VMEM((ECHUNK, H), jnp.float32),
            pltpu.VMEM((ECHUNK, H), jnp.float32),
            pltpu.VMEM((ECHUNK, H), jnp.float32),
            pltpu.VMEM((ECHUNK,), jnp.float32),
            pltpu.VMEM((ECHUNK,), jnp.float32),
            pltpu.VMEM((H, 16), jnp.float32),
            pltpu.VMEM((16,), jnp.float32),
            pltpu.SemaphoreType.DMA,
            pltpu.SemaphoreType.DMA,
            pltpu.SemaphoreType.DMA,
            pltpu.SemaphoreType.DMA,
        ],
    )(_edge_body)
    return f(a_tab, t_tab, eidx, w2b, b2b)


def kernel(x_agent, x_task, W_enc_a, b_enc_a, g_ln_a, be_ln_a, W_enc_t,
           b_enc_t, g_ln_t, be_ln_t, Wl_st, bl_st, Wr_st, Wl_ts, bl_ts,
           Wr_ts, W_h1, b_h1, W_h2, b_h2, edge_index):
    h_a, g_a = _encode(x_agent, W_enc_a, b_enc_a, g_ln_a, be_ln_a, NA)
    h_t, g_t = _encode(x_task, W_enc_t, b_enc_t, g_ln_t, be_ln_t, NT)
    zpad = jnp.zeros((2, NP - NA, RW), jnp.float32)
    g_a_flat = jnp.concatenate([g_a, zpad], axis=1).reshape(2 * NP, RW)
    g_t_flat = jnp.concatenate([g_t, zpad], axis=1).reshape(2 * NP, RW)

    src = edge_index[0]
    dst = edge_index[1]
    pad = E_PAD - E
    srcp = jnp.concatenate([src, jnp.full((pad,), NA, jnp.int32)])
    dstp = jnp.concatenate([dst, jnp.full((pad,), NA, jnp.int32)])
    src_raw = srcp.reshape(EROWS, 128)
    dst_raw = dstp.reshape(EROWS, 128)
    raws = jnp.stack([dst_raw, src_raw])
    src_off = jnp.stack([srcp, srcp + NP]).reshape(2, EROWS, 128)
    dst_off = jnp.stack([dstp, dstp + NP]).reshape(2, EROWS, 128)

    ones_hbm = jnp.ones((CCHUNK, CW), jnp.float32)
    zc_hbm = jnp.zeros((TPT, CW), jnp.float32)
    zf_hbm = jnp.zeros((TPT, RW), jnp.float32)

    cnt = _counts(raws, ones_hbm, zc_hbm)
    s_t, s_a = _feat(g_a_flat, g_t_flat, src_off, dst_off, dst_raw, src_raw,
                     zf_hbm)

    a_tab, t_tab = _post(s_t, s_a, cnt, h_a, h_t, Wl_st, bl_st, Wr_st,
                         Wl_ts, bl_ts, Wr_ts, W_h1, b_h1)

    src0 = jnp.concatenate([src, jnp.zeros((pad,), jnp.int32)])
    dst0 = jnp.concatenate([dst, jnp.zeros((pad,), jnp.int32)])
    eidx = jnp.stack([src0.reshape(EROWS, 128), dst0.reshape(EROWS, 128)])
    w2b = jnp.broadcast_to(W_h2[0][:, None], (H, 16))
    b2b = jnp.broadcast_to(b_h2, (16,))
    logits_pad = _edge_head(a_tab, t_tab, eidx, w2b, b2b)
    return logits_pad[:E]


# speedup vs baseline: 10.3753x; 1.7472x over previous
import functools

import jax
import jax.numpy as jnp
from jax import lax
from jax.experimental import pallas as pl
from jax.experimental.pallas import tpu as pltpu
from jax.experimental.pallas import tpu_sc as plsc

NA = 50000
NT = 50000
E = 800000
H = 64

RW = 32
CW = 16
NP = 50008
ACC_R = 50048
TPT = ACC_R // 16
E_PAD = 802816
EROWS = E_PAD // 128
TILE_EROWS = EROWS // 16
EH_TILE_EROWS = EROWS // 32

FKB = 2
FCHUNK = FKB * 128
NFCH = TILE_EROWS // FKB

CKB = 8
CCHUNK = CKB * 128
NCCH = TILE_EROWS // CKB

EKB = 2
ECHUNK = EKB * 128
NECH = EH_TILE_EROWS // EKB

ENC_R = 2000
POST_R = 2000

_SC_PARAMS = pltpu.CompilerParams(use_tc_tiling_on_sc=False,
                                  needs_layout_passes=False)


def _enc_body(x_ref, w_ref, b_ref, g_ref, be_ref, h_ref, gtab_ref):
    x = x_ref[...]
    y = jnp.dot(x, w_ref[...].T, preferred_element_type=jnp.float32)
    y = y + b_ref[...]
    m = jnp.mean(y, axis=-1, keepdims=True)
    v = jnp.mean((y - m) ** 2, axis=-1, keepdims=True)
    y = (y - m) * lax.rsqrt(v + 1e-5) * g_ref[...] + be_ref[...]
    h = jnp.maximum(y, 0.0)
    h_ref[...] = h
    gtab_ref[...] = jnp.stack([h[:, :RW], h[:, RW:]], axis=0)


def _encode(x, w, b, g, be, n):
    return pl.pallas_call(
        _enc_body,
        grid=(n // ENC_R,),
        in_specs=[
            pl.BlockSpec((ENC_R, 128), lambda i: (i, 0)),
            pl.BlockSpec((H, 128), lambda i: (0, 0)),
            pl.BlockSpec((H,), lambda i: (0,)),
            pl.BlockSpec((H,), lambda i: (0,)),
            pl.BlockSpec((H,), lambda i: (0,)),
        ],
        out_specs=[
            pl.BlockSpec((ENC_R, H), lambda i: (i, 0)),
            pl.BlockSpec((2, ENC_R, RW), lambda i: (0, i, 0)),
        ],
        out_shape=[
            jax.ShapeDtypeStruct((n, H), jnp.float32),
            jax.ShapeDtypeStruct((2, n, RW), jnp.float32),
        ],
    )(x, w, b, g, be)


def _cnt_body(raws, ones_hbm, zc_hbm, cnt_out, acc, sidx0, sidx1, ones_v, sem):
    c = lax.axis_index("c")
    s = lax.axis_index("s")
    pltpu.sync_copy(zc_hbm, acc.at[pl.ds(s * TPT, TPT)])
    pltpu.sync_copy(ones_hbm, ones_v)
    plsc.subcore_barrier()
    base = s * TILE_EROWS

    def stage(k, buf):
        pltpu.sync_copy(raws.at[c, pl.ds(base + k * CKB, CKB)], buf)

    def scat(buf):
        for j in range(CKB):
            pltpu.sync_copy(ones_v.at[pl.ds(j * 128, 128)],
                            acc.at[buf.at[j]], add=True)

    stage(0, sidx0)

    def body(i, _):
        stage(2 * i + 1, sidx1)
        scat(sidx0)
        stage(2 * i + 2, sidx0)
        scat(sidx1)
        return _

    lax.fori_loop(0, (NCCH - 1) // 2, body, None)
    scat(sidx0)
    plsc.subcore_barrier()
    pltpu.sync_copy(acc.at[pl.ds(s * TPT, TPT)],
                    cnt_out.at[c, pl.ds(s * TPT, TPT)])


def _counts(raws, ones_hbm, zc_hbm):
    mesh = plsc.VectorSubcoreMesh(core_axis_name="c", subcore_axis_name="s")
    f = functools.partial(
        pl.kernel,
        out_type=jax.ShapeDtypeStruct((2, ACC_R, CW), jnp.float32),
        mesh=mesh,
        compiler_params=_SC_PARAMS,
        scratch_types=[
            pltpu.VMEM_SHARED((ACC_R, CW), jnp.float32),
            pltpu.VMEM((CKB, 128), jnp.int32),
            pltpu.VMEM((CKB, 128), jnp.int32),
            pltpu.VMEM((CCHUNK, CW), jnp.float32),
            pltpu.SemaphoreType.DMA,
        ],
    )(_cnt_body)
    return f(raws, ones_hbm, zc_hbm)


def _feat_phase(c, s, table, goff, sraw, out, acc, gidx0, gidx1, sidx0, sidx1,
                rows0, rows1, sem0, sem1, zf_hbm):
    pltpu.sync_copy(zf_hbm, acc.at[pl.ds(s * TPT, TPT)])
    plsc.subcore_barrier()
    base = s * TILE_EROWS

    def stage_issue(k, gbuf, sbuf, rbuf, sem):
        r0 = base + k * FKB
        pltpu.sync_copy(goff.at[c, pl.ds(r0, FKB)], gbuf)
        pltpu.sync_copy(sraw.at[pl.ds(r0, FKB)], sbuf)
        for j in range(FKB):
            pltpu.async_copy(table.at[gbuf.at[j]],
                             rbuf.at[pl.ds(j * 128, 128)], sem)

    def wait(rbuf, sem):
        for j in range(FKB):
            pltpu.make_async_copy(table.at[pl.ds(0, 128)],
                                  rbuf.at[pl.ds(j * 128, 128)], sem).wait()

    def scat(rbuf, sbuf):
        for j in range(FKB):
            pltpu.sync_copy(rbuf.at[pl.ds(j * 128, 128)],
                            acc.at[sbuf.at[j]], add=True)

    stage_issue(0, gidx0, sidx0, rows0, sem0)

    def body(i, _):
        stage_issue(2 * i + 1, gidx1, sidx1, rows1, sem1)
        wait(rows0, sem0)
        scat(rows0, sidx0)
        stage_issue(2 * i + 2, gidx0, sidx0, rows0, sem0)
        wait(rows1, sem1)
        scat(rows1, sidx1)
        return _

    lax.fori_loop(0, (NFCH - 2) // 2, body, None)
    stage_issue(NFCH - 1, gidx1, sidx1, rows1, sem1)
    wait(rows0, sem0)
    scat(rows0, sidx0)
    wait(rows1, sem1)
    scat(rows1, sidx1)
    plsc.subcore_barrier()
    pltpu.sync_copy(acc.at[pl.ds(s * TPT, TPT)],
                    out.at[c, pl.ds(s * TPT, TPT)])
    plsc.subcore_barrier()


def _feat_body(g_a, g_t, src_off, dst_off, dst_raw, src_raw, zf_hbm,
               s_t, s_a, acc, gidx0, gidx1, sidx0, sidx1, rows0, rows1,
               sem0, sem1):
    c = lax.axis_index("c")
    s = lax.axis_index("s")
    _feat_phase(c, s, g_a, src_off, dst_raw, s_t, acc, gidx0, gidx1,
                sidx0, sidx1, rows0, rows1, sem0, sem1, zf_hbm)
    _feat_phase(c, s, g_t, dst_off, src_raw, s_a, acc, gidx0, gidx1,
                sidx0, sidx1, rows0, rows1, sem0, sem1, zf_hbm)


def _feat(g_a, g_t, src_off, dst_off, dst_raw, src_raw, zf_hbm):
    mesh = plsc.VectorSubcoreMesh(core_axis_name="c", subcore_axis_name="s")
    f = functools.partial(
        pl.kernel,
        out_type=[
            jax.ShapeDtypeStruct((2, ACC_R, RW), jnp.float32),
            jax.ShapeDtypeStruct((2, ACC_R, RW), jnp.float32),
        ],
        mesh=mesh,
        compiler_params=_SC_PARAMS,
        scratch_types=[
            pltpu.VMEM_SHARED((ACC_R, RW), jnp.float32),
            pltpu.VMEM((FKB, 128), jnp.int32),
            pltpu.VMEM((FKB, 128), jnp.int32),
            pltpu.VMEM((FKB, 128), jnp.int32),
            pltpu.VMEM((FKB, 128), jnp.int32),
            pltpu.VMEM((FCHUNK, RW), jnp.float32),
            pltpu.VMEM((FCHUNK, RW), jnp.float32),
            pltpu.SemaphoreType.DMA,
            pltpu.SemaphoreType.DMA,
        ],
    )(_feat_body)
    return f(g_a, g_t, src_off, dst_off, dst_raw, src_raw, zf_hbm)


def _post_body(st_ref, sa_ref, cnt_ref, ha_ref, ht_ref, wlst_ref, blst_ref,
               wrst_ref, wlts_ref, blts_ref, wrts_ref, wh1_ref, bh1_ref,
               a_ref, t_ref):
    st = st_ref[...]
    sa = sa_ref[...]
    cnt = cnt_ref[...]
    cnt_t = jnp.maximum(cnt[0, :, 0:1], 1.0)
    cnt_a = jnp.maximum(cnt[1, :, 0:1], 1.0)
    agg_t = jnp.concatenate([st[0], st[1]], axis=1) / cnt_t
    agg_a = jnp.concatenate([sa[0], sa[1]], axis=1) / cnt_a
    new_t = (jnp.dot(agg_t, wlst_ref[...].T, preferred_element_type=jnp.float32)
             + blst_ref[...]
             + jnp.dot(ht_ref[...], wrst_ref[...].T, preferred_element_type=jnp.float32))
    new_a = (jnp.dot(agg_a, wlts_ref[...].T, preferred_element_type=jnp.float32)
             + blts_ref[...]
             + jnp.dot(ha_ref[...], wrts_ref[...].T, preferred_element_type=jnp.float32))
    wh1 = wh1_ref[...]
    a_ref[...] = (jnp.dot(new_a, wh1[:, :H].T, preferred_element_type=jnp.float32)
                  + bh1_ref[...])
    t_ref[...] = jnp.dot(new_t, wh1[:, H:].T, preferred_element_type=jnp.float32)


def _post(s_t, s_a, cnt, h_a, h_t, wlst, blst, wrst, wlts, blts, wrts,
          wh1, bh1):
    return pl.pallas_call(
        _post_body,
        grid=(NA // POST_R,),
        in_specs=[
            pl.BlockSpec((2, POST_R, RW), lambda i: (0, i, 0)),
            pl.BlockSpec((2, POST_R, RW), lambda i: (0, i, 0)),
            pl.BlockSpec((2, POST_R, CW), lambda i: (0, i, 0)),
            pl.BlockSpec((POST_R, H), lambda i: (i, 0)),
            pl.BlockSpec((POST_R, H), lambda i: (i, 0)),
            pl.BlockSpec((H, H), lambda i: (0, 0)),
            pl.BlockSpec((H,), lambda i: (0,)),
            pl.BlockSpec((H, H), lambda i: (0, 0)),
            pl.BlockSpec((H, H), lambda i: (0, 0)),
            pl.BlockSpec((H,), lambda i: (0,)),
            pl.BlockSpec((H, H), lambda i: (0, 0)),
            pl.BlockSpec((H, 2 * H), lambda i: (0, 0)),
            pl.BlockSpec((H,), lambda i: (0,)),
        ],
        out_specs=[
            pl.BlockSpec((POST_R, H), lambda i: (i, 0)),
            pl.BlockSpec((POST_R, H), lambda i: (i, 0)),
        ],
        out_shape=[
            jax.ShapeDtypeStruct((NA, H), jnp.float32),
            jax.ShapeDtypeStruct((NT, H), jnp.float32),
        ],
    )(s_t, s_a, cnt, h_a, h_t, wlst, blst, wrst, wlts, blts, wrts, wh1, bh1)


def _edge_body(a_tab, t_tab, eidx, w2_hbm, b2_hbm, logits,
               aidx0, aidx1, tidx0, tidx1, arows0, arows1, trows0, trows1,
               out0, out1, w2_v, b2_v, tile, sg0, sg1, sw0, sw1):
    c = lax.axis_index("c")
    s = lax.axis_index("s")
    wid = s * 2 + c
    base = wid * EH_TILE_EROWS
    obase = wid * (EH_TILE_EROWS * 128)
    pltpu.sync_copy(w2_hbm, w2_v)
    pltpu.sync_copy(b2_hbm, b2_v)
    pltpu.async_copy(out0, logits.at[pl.ds(E, ECHUNK)], sw0)
    pltpu.async_copy(out1, logits.at[pl.ds(E, ECHUNK)], sw1)

    def stage_issue(k, aidx, tidx, arows, trows, sem):
        r0 = base + k * EKB
        pltpu.sync_copy(eidx.at[0, pl.ds(r0, EKB)], aidx)
        pltpu.sync_copy(eidx.at[1, pl.ds(r0, EKB)], tidx)
        for j in range(EKB):
            pltpu.async_copy(a_tab.at[aidx.at[j]],
                             arows.at[pl.ds(j * 128, 128)], sem)
            pltpu.async_copy(t_tab.at[tidx.at[j]],
                             trows.at[pl.ds(j * 128, 128)], sem)

    def wait_pair(arows, trows, sem):
        for j in range(EKB):
            pltpu.make_async_copy(a_tab.at[pl.ds(0, 128)],
                                  arows.at[pl.ds(j * 128, 128)], sem).wait()
            pltpu.make_async_copy(t_tab.at[pl.ds(0, 128)],
                                  trows.at[pl.ds(j * 128, 128)], sem).wait()

    w2v = [w2_v[pl.ds(q * 16, 16)] for q in range(4)]
    b2vec = b2_v[...]
    iot = lax.iota(jnp.int32, 16)

    def compute(k, arows, trows, out, sw):
        pltpu.make_async_copy(out, logits.at[pl.ds(E, ECHUNK)], sw).wait()

        def group(g, _):
            e0 = g * 16
            for u in range(16):
                e = e0 + u
                t = None
                for q in range(4):
                    av = arows[e, pl.ds(q * 16, 16)]
                    tv = trows[e, pl.ds(q * 16, 16)]
                    r = jnp.maximum(av + tv, 0.0) * w2v[q]
                    t = r if t is None else t + r
                tile[u, :] = t
            res = b2vec
            for q in range(16):
                col = plsc.load_gather(tile, [iot, jnp.full((16,), q, jnp.int32)])
                res = res + col
            out[pl.ds(e0, 16)] = res
            return _

        lax.fori_loop(0, ECHUNK // 16, group, None)
        pltpu.async_copy(out, logits.at[pl.ds(obase + k * ECHUNK, ECHUNK)], sw)

    stage_issue(0, aidx0, tidx0, arows0, trows0, sg0)

    def body(i, _):
        stage_issue(2 * i + 1, aidx1, tidx1, arows1, trows1, sg1)
        wait_pair(arows0, trows0, sg0)
        compute(2 * i, arows0, trows0, out0, sw0)
        stage_issue(2 * i + 2, aidx0, tidx0, arows0, trows0, sg0)
        wait_pair(arows1, trows1, sg1)
        compute(2 * i + 1, arows1, trows1, out1, sw1)
        return _

    lax.fori_loop(0, (NECH - 2) // 2, body, None)
    stage_issue(NECH - 1, aidx1, tidx1, arows1, trows1, sg1)
    wait_pair(arows0, trows0, sg0)
    compute(NECH - 2, arows0, trows0, out0, sw0)
    wait_pair(arows1, trows1, sg1)
    compute(NECH - 1, arows1, trows1, out1, sw1)
    pltpu.make_async_copy(out0, logits.at[pl.ds(E, ECHUNK)], sw0).wait()
    pltpu.make_async_copy(out1, logits.at[pl.ds(E, ECHUNK)], sw1).wait()


def _edge_head(a_tab, t_tab, eidx, w2b, b2b):
    mesh = plsc.VectorSubcoreMesh(core_axis_name="c", subcore_axis_name="s")
    f = functools.partial(
        pl.kernel,
        out_type=jax.ShapeDtypeStruct((E_PAD,), jnp.float32),
        mesh=mesh,
        compiler_params=_SC_PARAMS,
        scratch_types=[
            pltpu.VMEM((EKB, 128), jnp.int32),
            pltpu.VMEM((EKB, 128), jnp.int32),
            pltpu.VMEM((EKB, 128), jnp.int32),
            pltpu.VMEM((EKB, 128), jnp.int32),
            pltpu.VMEM((ECHUNK, H), jnp.float32),
            pltpu.VMEM((ECHUNK, H), jnp.float32),
            pltpu.VMEM((ECHUNK, H), jnp.float32),
            pltpu.VMEM((ECHUNK, H), jnp.float32),
            pltpu.VMEM((ECHUNK,), jnp.float32),
            pltpu.VMEM((ECHUNK,), jnp.float32),
            pltpu.VMEM((H,), jnp.float32),
            pltpu.VMEM((16,), jnp.float32),
            pltpu.VMEM((16, 16), jnp.float32),
            pltpu.SemaphoreType.DMA,
            pltpu.SemaphoreType.DMA,
            pltpu.SemaphoreType.DMA,
            pltpu.SemaphoreType.DMA,
        ],
    )(_edge_body)
    return f(a_tab, t_tab, eidx, w2b, b2b)


def kernel(x_agent, x_task, W_enc_a, b_enc_a, g_ln_a, be_ln_a, W_enc_t,
           b_enc_t, g_ln_t, be_ln_t, Wl_st, bl_st, Wr_st, Wl_ts, bl_ts,
           Wr_ts, W_h1, b_h1, W_h2, b_h2, edge_index):
    h_a, g_a = _encode(x_agent, W_enc_a, b_enc_a, g_ln_a, be_ln_a, NA)
    h_t, g_t = _encode(x_task, W_enc_t, b_enc_t, g_ln_t, be_ln_t, NT)
    zpad = jnp.zeros((2, NP - NA, RW), jnp.float32)
    g_a_flat = jnp.concatenate([g_a, zpad], axis=1).reshape(2 * NP, RW)
    g_t_flat = jnp.concatenate([g_t, zpad], axis=1).reshape(2 * NP, RW)

    src = edge_index[0]
    dst = edge_index[1]
    pad = E_PAD - E
    srcp = jnp.concatenate([src, jnp.full((pad,), NA, jnp.int32)])
    dstp = jnp.concatenate([dst, jnp.full((pad,), NA, jnp.int32)])
    src_raw = srcp.reshape(EROWS, 128)
    dst_raw = dstp.reshape(EROWS, 128)
    raws = jnp.stack([dst_raw, src_raw])
    src_off = jnp.stack([srcp, srcp + NP]).reshape(2, EROWS, 128)
    dst_off = jnp.stack([dstp, dstp + NP]).reshape(2, EROWS, 128)

    ones_hbm = jnp.ones((CCHUNK, CW), jnp.float32)
    zc_hbm = jnp.zeros((TPT, CW), jnp.float32)
    zf_hbm = jnp.zeros((TPT, RW), jnp.float32)

    cnt = _counts(raws, ones_hbm, zc_hbm)
    s_t, s_a = _feat(g_a_flat, g_t_flat, src_off, dst_off, dst_raw, src_raw,
                     zf_hbm)

    a_tab, t_tab = _post(s_t, s_a, cnt, h_a, h_t, Wl_st, bl_st, Wr_st,
                         Wl_ts, bl_ts, Wr_ts, W_h1, b_h1)

    src0 = jnp.concatenate([src, jnp.zeros((pad,), jnp.int32)])
    dst0 = jnp.concatenate([dst, jnp.zeros((pad,), jnp.int32)])
    eidx = jnp.stack([src0.reshape(EROWS, 128), dst0.reshape(EROWS, 128)])
    w2b = W_h2[0]
    b2b = jnp.broadcast_to(b_h2, (16,))
    logits_pad = _edge_head(a_tab, t_tab, eidx, w2b, b2b)
    return logits_pad[:E]


# speedup vs baseline: 11.2574x; 1.0850x over previous
import functools

import jax
import jax.numpy as jnp
from jax import lax
from jax.experimental import pallas as pl
from jax.experimental.pallas import tpu as pltpu
from jax.experimental.pallas import tpu_sc as plsc

NA = 50000
NT = 50000
E = 800000
H = 64

RW = 32
CW = 16
NP = 50008
ACC_R = 50048
TPT = ACC_R // 16
E_PAD = 802816
EROWS = E_PAD // 128
TILE_EROWS = EROWS // 16
EH_TILE_EROWS = EROWS // 32

FKB = 2
FCHUNK = FKB * 128
NFCH = TILE_EROWS // FKB

CKB = 8
CCHUNK = CKB * 128
NCCH = TILE_EROWS // CKB

EKB = 2
ECHUNK = EKB * 128
NECH = EH_TILE_EROWS // EKB

ENC_R = 2000
POST_R = 2000

_SC_PARAMS = pltpu.CompilerParams(use_tc_tiling_on_sc=False,
                                  needs_layout_passes=False)


def _enc_body(x_ref, w_ref, b_ref, g_ref, be_ref, h_ref, gtab_ref):
    x = x_ref[...]
    y = jnp.dot(x, w_ref[...].T, preferred_element_type=jnp.float32)
    y = y + b_ref[...]
    m = jnp.mean(y, axis=-1, keepdims=True)
    v = jnp.mean((y - m) ** 2, axis=-1, keepdims=True)
    y = (y - m) * lax.rsqrt(v + 1e-5) * g_ref[...] + be_ref[...]
    h = jnp.maximum(y, 0.0)
    h_ref[...] = h
    gtab_ref[...] = jnp.stack([h[:, :RW], h[:, RW:]], axis=0)


def _encode(x, w, b, g, be, n):
    return pl.pallas_call(
        _enc_body,
        grid=(n // ENC_R,),
        in_specs=[
            pl.BlockSpec((ENC_R, 128), lambda i: (i, 0)),
            pl.BlockSpec((H, 128), lambda i: (0, 0)),
            pl.BlockSpec((H,), lambda i: (0,)),
            pl.BlockSpec((H,), lambda i: (0,)),
            pl.BlockSpec((H,), lambda i: (0,)),
        ],
        out_specs=[
            pl.BlockSpec((ENC_R, H), lambda i: (i, 0)),
            pl.BlockSpec((2, ENC_R, RW), lambda i: (0, i, 0)),
        ],
        out_shape=[
            jax.ShapeDtypeStruct((n, H), jnp.float32),
            jax.ShapeDtypeStruct((2, n, RW), jnp.float32),
        ],
    )(x, w, b, g, be)


def _cnt_body(raws, ones_hbm, zc_hbm, cnt_out, acc, sidx0, sidx1, ones_v, sem):
    c = lax.axis_index("c")
    s = lax.axis_index("s")
    pltpu.sync_copy(zc_hbm, acc.at[pl.ds(s * TPT, TPT)])
    pltpu.sync_copy(ones_hbm, ones_v)
    plsc.subcore_barrier()
    base = s * TILE_EROWS

    def stage(k, buf):
        pltpu.sync_copy(raws.at[c, pl.ds(base + k * CKB, CKB)], buf)

    def scat(buf):
        for j in range(CKB):
            pltpu.sync_copy(ones_v.at[pl.ds(j * 128, 128)],
                            acc.at[buf.at[j]], add=True)

    stage(0, sidx0)

    def body(i, _):
        stage(2 * i + 1, sidx1)
        scat(sidx0)
        stage(2 * i + 2, sidx0)
        scat(sidx1)
        return _

    lax.fori_loop(0, (NCCH - 1) // 2, body, None)
    scat(sidx0)
    plsc.subcore_barrier()
    pltpu.sync_copy(acc.at[pl.ds(s * TPT, TPT)],
                    cnt_out.at[c, pl.ds(s * TPT, TPT)])


def _counts(raws, ones_hbm, zc_hbm):
    mesh = plsc.VectorSubcoreMesh(core_axis_name="c", subcore_axis_name="s")
    f = functools.partial(
        pl.kernel,
        out_type=jax.ShapeDtypeStruct((2, ACC_R, CW), jnp.float32),
        mesh=mesh,
        compiler_params=_SC_PARAMS,
        scratch_types=[
            pltpu.VMEM_SHARED((ACC_R, CW), jnp.float32),
            pltpu.VMEM((CKB, 128), jnp.int32),
            pltpu.VMEM((CKB, 128), jnp.int32),
            pltpu.VMEM((CCHUNK, CW), jnp.float32),
            pltpu.SemaphoreType.DMA,
        ],
    )(_cnt_body)
    return f(raws, ones_hbm, zc_hbm)


def _feat_phase(c, s, table, comb, out, acc, cidx, rows, gsems, ssems,
                zf_hbm):
    pltpu.sync_copy(zf_hbm, acc.at[pl.ds(s * TPT, TPT)])
    plsc.subcore_barrier()
    base = s * TILE_EROWS

    def stage_issue(k, b):
        r0 = 2 * (base + k * FKB)
        pltpu.sync_copy(comb.at[c, pl.ds(r0, 2 * FKB)], cidx[b])
        for j in range(FKB):
            pltpu.async_copy(table.at[cidx[b].at[2 * j]],
                             rows[b].at[pl.ds(j * 128, 128)], gsems[b])

    def wait_gather(b):
        for j in range(FKB):
            pltpu.make_async_copy(table.at[pl.ds(0, 128)],
                                  rows[b].at[pl.ds(j * 128, 128)],
                                  gsems[b]).wait()

    def issue_scatter(b):
        for j in range(FKB):
            pltpu.async_copy(rows[b].at[pl.ds(j * 128, 128)],
                             acc.at[cidx[b].at[2 * j + 1]], ssems[b],
                             add=True)

    def wait_scatter(b):
        for j in range(FKB):
            pltpu.make_async_copy(rows[b].at[pl.ds(j * 128, 128)],
                                  acc.at[pl.ds(0, 128)], ssems[b]).wait()

    def S(k, b):
        wait_gather(b)
        issue_scatter(b)

    def G(k, b):
        wait_scatter(b)
        stage_issue(k, b)

    for k in range(3):
        stage_issue(k, k)

    def body(jj, _):
        k0 = 3 * jj
        for u in range(3):
            S(k0 + u, u)
            G(k0 + u + 3, u)
        return _

    NLOOP = (NFCH - 4) // 3
    lax.fori_loop(0, NLOOP, body, None)
    kk = 3 * NLOOP
    S(kk, kk % 3)
    G(NFCH - 1, (NFCH - 1) % 3)
    S(kk + 1, (kk + 1) % 3)
    S(kk + 2, (kk + 2) % 3)
    S(NFCH - 1, (NFCH - 1) % 3)
    for b in range(3):
        wait_scatter(b)
    plsc.subcore_barrier()
    pltpu.sync_copy(acc.at[pl.ds(s * TPT, TPT)],
                    out.at[c, pl.ds(s * TPT, TPT)])
    plsc.subcore_barrier()


def _feat_body(g_a, g_t, comb_t, comb_a, zf_hbm,
               s_t, s_a, acc, cidx0, cidx1, cidx2, rows0, rows1, rows2,
               gs0, gs1, gs2, ss0, ss1, ss2):
    c = lax.axis_index("c")
    s = lax.axis_index("s")
    cidx = (cidx0, cidx1, cidx2)
    rows = (rows0, rows1, rows2)
    gsems = (gs0, gs1, gs2)
    ssems = (ss0, ss1, ss2)
    _feat_phase(c, s, g_a, comb_t, s_t, acc, cidx, rows, gsems, ssems,
                zf_hbm)
    _feat_phase(c, s, g_t, comb_a, s_a, acc, cidx, rows, gsems, ssems,
                zf_hbm)


def _feat(g_a, g_t, comb_t, comb_a, zf_hbm):
    mesh = plsc.VectorSubcoreMesh(core_axis_name="c", subcore_axis_name="s")
    f = functools.partial(
        pl.kernel,
        out_type=[
            jax.ShapeDtypeStruct((2, ACC_R, RW), jnp.float32),
            jax.ShapeDtypeStruct((2, ACC_R, RW), jnp.float32),
        ],
        mesh=mesh,
        compiler_params=_SC_PARAMS,
        scratch_types=[
            pltpu.VMEM_SHARED((ACC_R, RW), jnp.float32),
            pltpu.VMEM((2 * FKB, 128), jnp.int32),
            pltpu.VMEM((2 * FKB, 128), jnp.int32),
            pltpu.VMEM((2 * FKB, 128), jnp.int32),
            pltpu.VMEM((FCHUNK, RW), jnp.float32),
            pltpu.VMEM((FCHUNK, RW), jnp.float32),
            pltpu.VMEM((FCHUNK, RW), jnp.float32),
            pltpu.SemaphoreType.DMA,
            pltpu.SemaphoreType.DMA,
            pltpu.SemaphoreType.DMA,
            pltpu.SemaphoreType.DMA,
            pltpu.SemaphoreType.DMA,
            pltpu.SemaphoreType.DMA,
        ],
    )(_feat_body)
    return f(g_a, g_t, comb_t, comb_a, zf_hbm)


def _post_body(st_ref, sa_ref, cnt_ref, ha_ref, ht_ref, wlst_ref, blst_ref,
               wrst_ref, wlts_ref, blts_ref, wrts_ref, wh1_ref, bh1_ref,
               a_ref, t_ref):
    st = st_ref[...]
    sa = sa_ref[...]
    cnt = cnt_ref[...]
    cnt_t = jnp.maximum(cnt[0, :, 0:1], 1.0)
    cnt_a = jnp.maximum(cnt[1, :, 0:1], 1.0)
    agg_t = jnp.concatenate([st[0], st[1]], axis=1) / cnt_t
    agg_a = jnp.concatenate([sa[0], sa[1]], axis=1) / cnt_a
    new_t = (jnp.dot(agg_t, wlst_ref[...].T, preferred_element_type=jnp.float32)
             + blst_ref[...]
             + jnp.dot(ht_ref[...], wrst_ref[...].T, preferred_element_type=jnp.float32))
    new_a = (jnp.dot(agg_a, wlts_ref[...].T, preferred_element_type=jnp.float32)
             + blts_ref[...]
             + jnp.dot(ha_ref[...], wrts_ref[...].T, preferred_element_type=jnp.float32))
    wh1 = wh1_ref[...]
    a_ref[...] = (jnp.dot(new_a, wh1[:, :H].T, preferred_element_type=jnp.float32)
                  + bh1_ref[...])
    t_ref[...] = jnp.dot(new_t, wh1[:, H:].T, preferred_element_type=jnp.float32)


def _post(s_t, s_a, cnt, h_a, h_t, wlst, blst, wrst, wlts, blts, wrts,
          wh1, bh1):
    return pl.pallas_call(
        _post_body,
        grid=(NA // POST_R,),
        in_specs=[
            pl.BlockSpec((2, POST_R, RW), lambda i: (0, i, 0)),
            pl.BlockSpec((2, POST_R, RW), lambda i: (0, i, 0)),
            pl.BlockSpec((2, POST_R, CW), lambda i: (0, i, 0)),
            pl.BlockSpec((POST_R, H), lambda i: (i, 0)),
            pl.BlockSpec((POST_R, H), lambda i: (i, 0)),
            pl.BlockSpec((H, H), lambda i: (0, 0)),
            pl.BlockSpec((H,), lambda i: (0,)),
            pl.BlockSpec((H, H), lambda i: (0, 0)),
            pl.BlockSpec((H, H), lambda i: (0, 0)),
            pl.BlockSpec((H,), lambda i: (0,)),
            pl.BlockSpec((H, H), lambda i: (0, 0)),
            pl.BlockSpec((H, 2 * H), lambda i: (0, 0)),
            pl.BlockSpec((H,), lambda i: (0,)),
        ],
        out_specs=[
            pl.BlockSpec((POST_R, H), lambda i: (i, 0)),
            pl.BlockSpec((POST_R, H), lambda i: (i, 0)),
        ],
        out_shape=[
            jax.ShapeDtypeStruct((NA, H), jnp.float32),
            jax.ShapeDtypeStruct((NT, H), jnp.float32),
        ],
    )(s_t, s_a, cnt, h_a, h_t, wlst, blst, wrst, wlts, blts, wrts, wh1, bh1)


def _edge_body(a_tab, t_tab, eidx, w2_hbm, b2_hbm, logits,
               aidx0, aidx1, tidx0, tidx1, arows0, arows1, trows0, trows1,
               out0, out1, w2_v, b2_v, tile, sg0, sg1, sw0, sw1):
    c = lax.axis_index("c")
    s = lax.axis_index("s")
    wid = s * 2 + c
    base = wid * EH_TILE_EROWS
    obase = wid * (EH_TILE_EROWS * 128)
    pltpu.sync_copy(w2_hbm, w2_v)
    pltpu.sync_copy(b2_hbm, b2_v)
    pltpu.async_copy(out0, logits.at[pl.ds(E, ECHUNK)], sw0)
    pltpu.async_copy(out1, logits.at[pl.ds(E, ECHUNK)], sw1)

    def stage_issue(k, aidx, tidx, arows, trows, sem):
        r0 = base + k * EKB
        pltpu.sync_copy(eidx.at[0, pl.ds(r0, EKB)], aidx)
        pltpu.sync_copy(eidx.at[1, pl.ds(r0, EKB)], tidx)
        for j in range(EKB):
            pltpu.async_copy(a_tab.at[aidx.at[j]],
                             arows.at[pl.ds(j * 128, 128)], sem)
            pltpu.async_copy(t_tab.at[tidx.at[j]],
                             trows.at[pl.ds(j * 128, 128)], sem)

    def wait_pair(arows, trows, sem):
        for j in range(EKB):
            pltpu.make_async_copy(a_tab.at[pl.ds(0, 128)],
                                  arows.at[pl.ds(j * 128, 128)], sem).wait()
            pltpu.make_async_copy(t_tab.at[pl.ds(0, 128)],
                                  trows.at[pl.ds(j * 128, 128)], sem).wait()

    w2v = [w2_v[pl.ds(q * 16, 16)] for q in range(4)]
    b2vec = b2_v[...]
    iot = lax.iota(jnp.int32, 16)

    def compute(k, arows, trows, out, sw):
        pltpu.make_async_copy(out, logits.at[pl.ds(E, ECHUNK)], sw).wait()

        def group(g, _):
            e0 = g * 16
            for u in range(16):
                e = e0 + u
                t = None
                for q in range(4):
                    av = arows[e, pl.ds(q * 16, 16)]
                    tv = trows[e, pl.ds(q * 16, 16)]
                    r = jnp.maximum(av + tv, 0.0) * w2v[q]
                    t = r if t is None else t + r
                tile[u, :] = t
            res = b2vec
            for q in range(16):
                col = plsc.load_gather(tile, [iot, jnp.full((16,), q, jnp.int32)])
                res = res + col
            out[pl.ds(e0, 16)] = res
            return _

        lax.fori_loop(0, ECHUNK // 16, group, None)
        pltpu.async_copy(out, logits.at[pl.ds(obase + k * ECHUNK, ECHUNK)], sw)

    stage_issue(0, aidx0, tidx0, arows0, trows0, sg0)

    def body(i, _):
        stage_issue(2 * i + 1, aidx1, tidx1, arows1, trows1, sg1)
        wait_pair(arows0, trows0, sg0)
        compute(2 * i, arows0, trows0, out0, sw0)
        stage_issue(2 * i + 2, aidx0, tidx0, arows0, trows0, sg0)
        wait_pair(arows1, trows1, sg1)
        compute(2 * i + 1, arows1, trows1, out1, sw1)
        return _

    lax.fori_loop(0, (NECH - 2) // 2, body, None)
    stage_issue(NECH - 1, aidx1, tidx1, arows1, trows1, sg1)
    wait_pair(arows0, trows0, sg0)
    compute(NECH - 2, arows0, trows0, out0, sw0)
    wait_pair(arows1, trows1, sg1)
    compute(NECH - 1, arows1, trows1, out1, sw1)
    pltpu.make_async_copy(out0, logits.at[pl.ds(E, ECHUNK)], sw0).wait()
    pltpu.make_async_copy(out1, logits.at[pl.ds(E, ECHUNK)], sw1).wait()


def _edge_head(a_tab, t_tab, eidx, w2b, b2b):
    mesh = plsc.VectorSubcoreMesh(core_axis_name="c", subcore_axis_name="s")
    f = functools.partial(
        pl.kernel,
        out_type=jax.ShapeDtypeStruct((E_PAD,), jnp.float32),
        mesh=mesh,
        compiler_params=_SC_PARAMS,
        scratch_types=[
            pltpu.VMEM((EKB, 128), jnp.int32),
            pltpu.VMEM((EKB, 128), jnp.int32),
            pltpu.VMEM((EKB, 128), jnp.int32),
            pltpu.VMEM((EKB, 128), jnp.int32),
            pltpu.VMEM((ECHUNK, H), jnp.float32),
            pltpu.VMEM((ECHUNK, H), jnp.float32),
            pltpu.VMEM((ECHUNK, H), jnp.float32),
            pltpu.VMEM((ECHUNK, H), jnp.float32),
            pltpu.VMEM((ECHUNK,), jnp.float32),
            pltpu.VMEM((ECHUNK,), jnp.float32),
            pltpu.VMEM((H,), jnp.float32),
            pltpu.VMEM((16,), jnp.float32),
            pltpu.VMEM((16, 16), jnp.float32),
            pltpu.SemaphoreType.DMA,
            pltpu.SemaphoreType.DMA,
            pltpu.SemaphoreType.DMA,
            pltpu.SemaphoreType.DMA,
        ],
    )(_edge_body)
    return f(a_tab, t_tab, eidx, w2b, b2b)


def kernel(x_agent, x_task, W_enc_a, b_enc_a, g_ln_a, be_ln_a, W_enc_t,
           b_enc_t, g_ln_t, be_ln_t, Wl_st, bl_st, Wr_st, Wl_ts, bl_ts,
           Wr_ts, W_h1, b_h1, W_h2, b_h2, edge_index):
    h_a, g_a = _encode(x_agent, W_enc_a, b_enc_a, g_ln_a, be_ln_a, NA)
    h_t, g_t = _encode(x_task, W_enc_t, b_enc_t, g_ln_t, be_ln_t, NT)
    zpad = jnp.zeros((2, NP - NA, RW), jnp.float32)
    g_a_flat = jnp.concatenate([g_a, zpad], axis=1).reshape(2 * NP, RW)
    g_t_flat = jnp.concatenate([g_t, zpad], axis=1).reshape(2 * NP, RW)

    src = edge_index[0]
    dst = edge_index[1]
    pad = E_PAD - E
    srcp = jnp.concatenate([src, jnp.full((pad,), NA, jnp.int32)])
    dstp = jnp.concatenate([dst, jnp.full((pad,), NA, jnp.int32)])
    src_raw = srcp.reshape(EROWS, 128)
    dst_raw = dstp.reshape(EROWS, 128)
    raws = jnp.stack([dst_raw, src_raw])
    src_off = jnp.stack([srcp, srcp + NP]).reshape(2, EROWS, 128)
    dst_off = jnp.stack([dstp, dstp + NP]).reshape(2, EROWS, 128)

    ones_hbm = jnp.ones((CCHUNK, CW), jnp.float32)
    zc_hbm = jnp.zeros((TPT, CW), jnp.float32)
    zf_hbm = jnp.zeros((TPT, RW), jnp.float32)

    cnt = _counts(raws, ones_hbm, zc_hbm)
    dst2 = jnp.broadcast_to(dst_raw, (2, EROWS, 128))
    src2 = jnp.broadcast_to(src_raw, (2, EROWS, 128))
    comb_t = jnp.stack([src_off, dst2], axis=2).reshape(2, 2 * EROWS, 128)
    comb_a = jnp.stack([dst_off, src2], axis=2).reshape(2, 2 * EROWS, 128)
    s_t, s_a = _feat(g_a_flat, g_t_flat, comb_t, comb_a, zf_hbm)

    a_tab, t_tab = _post(s_t, s_a, cnt, h_a, h_t, Wl_st, bl_st, Wr_st,
                         Wl_ts, bl_ts, Wr_ts, W_h1, b_h1)

    src0 = jnp.concatenate([src, jnp.zeros((pad,), jnp.int32)])
    dst0 = jnp.concatenate([dst, jnp.zeros((pad,), jnp.int32)])
    eidx = jnp.stack([src0.reshape(EROWS, 128), dst0.reshape(EROWS, 128)])
    w2b = W_h2[0]
    b2b = jnp.broadcast_to(b_h2, (16,))
    logits_pad = _edge_head(a_tab, t_tab, eidx, w2b, b2b)
    return logits_pad[:E]
